# baseline (speedup 1.0000x reference)
import functools

import jax
import jax.numpy as jnp
from jax import lax
from jax.experimental import pallas as pl
from jax.experimental.pallas import tpu as pltpu
from jax.experimental.pallas import tpu_sc as plsc

NTOKEN = 100000
EMB_DIM = 64
BATCH = 4096
SEQ = 50

NC, NS = 2, 16
NW = NC * NS
BPW = BATCH // NW
SEQP = 56
CHUNK = 2 * SEQP
NCHUNK = BPW // 2
NBUF = 8
ROUNDS = NCHUNK // NBUF


def _embed2(x3, Wcat):
    mesh = plsc.VectorSubcoreMesh(core_axis_name="c", subcore_axis_name="s")

    @functools.partial(
        pl.kernel,
        mesh=mesh,
        out_type=jax.ShapeDtypeStruct((BATCH, SEQ, 2 * EMB_DIM), jnp.float32),
        scratch_types=[
            pltpu.VMEM((NCHUNK, CHUNK), jnp.int32),
            *[pltpu.VMEM((CHUNK, 2 * EMB_DIM), jnp.float32)
              for _ in range(NBUF)],
            *[pltpu.SemaphoreType.DMA for _ in range(3 * NBUF)],
        ],
    )
    def k(x_hbm, w_hbm, out_hbm, idx_v, *rest):
        combs = rest[:NBUF]
        sgs = rest[NBUF:2 * NBUF]
        so1 = rest[2 * NBUF:3 * NBUF]
        so2 = rest[3 * NBUF:]
        wid = lax.axis_index("s") * NC + lax.axis_index("c")
        b0 = wid * BPW
        pltpu.sync_copy(x_hbm.at[wid], idx_v)

        def gather(j, b):
            pltpu.async_copy(w_hbm.at[idx_v.at[j]], combs[b], sgs[b])

        def wait_gather(j, b):
            pltpu.make_async_copy(
                w_hbm.at[idx_v.at[j]], combs[b], sgs[b]).wait()

        def store(j, b):
            bb = b0 + 2 * j
            pltpu.async_copy(
                combs[b].at[pl.ds(0, SEQ)], out_hbm.at[bb], so1[b])
            pltpu.async_copy(
                combs[b].at[pl.ds(SEQP, SEQ)], out_hbm.at[bb + 1], so2[b])

        def wait_store(j, b):
            bb = b0 + 2 * j
            pltpu.make_async_copy(
                combs[b].at[pl.ds(0, SEQ)], out_hbm.at[bb], so1[b]).wait()
            pltpu.make_async_copy(
                combs[b].at[pl.ds(SEQP, SEQ)], out_hbm.at[bb + 1],
                so2[b]).wait()

        for b in range(NBUF):
            gather(b, b)

        def outer(r, carry):
            for b in range(NBUF):
                j = r * NBUF + b
                wait_gather(j, b)
                store(j, b)
            for b in range(NBUF):
                j = r * NBUF + b
                wait_store(j, b)
                gather(j + NBUF, b)
            return carry

        lax.fori_loop(0, ROUNDS - 1, outer, 0)

        last = (ROUNDS - 1) * NBUF
        for b in range(NBUF):
            wait_gather(last + b, b)
            store(last + b, b)
        for b in range(NBUF):
            wait_store(last + b, b)

    return k(x3, Wcat)


def kernel(x, W, W_):
    Wcat = jnp.concatenate([W, W_], axis=1)
    xi = x.astype(jnp.int32)
    xp = jnp.concatenate([xi, xi[:, :SEQP - SEQ]], axis=1)
    x3 = xp.reshape(NW, NCHUNK, CHUNK)
    return _embed2(x3, Wcat)


# speedup vs baseline: 1.0072x; 1.0072x over previous
import functools

import jax
import jax.numpy as jnp
from jax import lax
from jax.experimental import pallas as pl
from jax.experimental.pallas import tpu as pltpu
from jax.experimental.pallas import tpu_sc as plsc

NTOKEN = 100000
EMB_DIM = 64
BATCH = 4096
SEQ = 50

NC, NS = 2, 16
NW = NC * NS
BPW = BATCH // NW
SEQP = 56
CHUNK = 2 * SEQP
NCHUNK = BPW // 2
NBUF = 4
ROUNDS = NCHUNK // NBUF


def _embed2(x3, Wcat):
    mesh = plsc.VectorSubcoreMesh(core_axis_name="c", subcore_axis_name="s")

    @functools.partial(
        pl.kernel,
        mesh=mesh,
        out_type=jax.ShapeDtypeStruct((BATCH, SEQ, 2 * EMB_DIM), jnp.float32),
        scratch_types=[
            pltpu.VMEM((NCHUNK, CHUNK), jnp.int32),
            *[pltpu.VMEM((CHUNK, 2 * EMB_DIM), jnp.float32)
              for _ in range(NBUF)],
            *[pltpu.SemaphoreType.DMA for _ in range(3 * NBUF)],
        ],
    )
    def k(x_hbm, w_hbm, out_hbm, idx_v, *rest):
        combs = rest[:NBUF]
        sgs = rest[NBUF:2 * NBUF]
        so1 = rest[2 * NBUF:3 * NBUF]
        so2 = rest[3 * NBUF:]
        wid = lax.axis_index("s") * NC + lax.axis_index("c")
        b0 = wid * BPW
        pltpu.sync_copy(x_hbm.at[wid], idx_v)

        def gather(j, b):
            pltpu.async_copy(w_hbm.at[idx_v.at[j]], combs[b], sgs[b])

        def wait_gather(j, b):
            pltpu.make_async_copy(
                w_hbm.at[idx_v.at[j]], combs[b], sgs[b]).wait()

        def store(j, b):
            bb = b0 + 2 * j
            pltpu.async_copy(
                combs[b].at[pl.ds(0, SEQ)], out_hbm.at[bb], so1[b])
            pltpu.async_copy(
                combs[b].at[pl.ds(SEQP, SEQ)], out_hbm.at[bb + 1], so2[b])

        def wait_store(j, b):
            bb = b0 + 2 * j
            pltpu.make_async_copy(
                combs[b].at[pl.ds(0, SEQ)], out_hbm.at[bb], so1[b]).wait()
            pltpu.make_async_copy(
                combs[b].at[pl.ds(SEQP, SEQ)], out_hbm.at[bb + 1],
                so2[b]).wait()

        for b in range(NBUF):
            gather(b, b)

        def outer(r, carry):
            for b in range(NBUF):
                j = r * NBUF + b
                wait_gather(j, b)
                store(j, b)
            for b in range(NBUF):
                j = r * NBUF + b
                wait_store(j, b)
                gather(j + NBUF, b)
            return carry

        lax.fori_loop(0, ROUNDS - 1, outer, 0)

        last = (ROUNDS - 1) * NBUF
        for b in range(NBUF):
            wait_gather(last + b, b)
            store(last + b, b)
        for b in range(NBUF):
            wait_store(last + b, b)

    return k(x3, Wcat)


def kernel(x, W, W_):
    Wcat = jnp.concatenate([W, W_], axis=1)
    xi = x.astype(jnp.int32)
    xp = jnp.concatenate([xi, xi[:, :SEQP - SEQ]], axis=1)
    x3 = xp.reshape(NW, NCHUNK, CHUNK)
    return _embed2(x3, Wcat)
